# Initial kernel scaffold; baseline (speedup 1.0000x reference)
import jax
import jax.numpy as jnp
from jax.experimental import pallas as pl


def kernel(x, table):
    raise NotImplementedError("write your pallas kernel here")


# speedup vs baseline: 4.0871x; 4.0871x over previous
import functools

import jax
import jax.numpy as jnp
from jax import lax
from jax.experimental import pallas as pl
from jax.experimental.pallas import tpu as pltpu
from jax.experimental.pallas import tpu_sc as plsc

EMBED_DIM = 64
CHUNK = 128


@functools.cache
def _make_kernel(b_flat):
    info = plsc.get_sparse_core_info()
    num_cores, num_subcores = info.num_cores, info.num_subcores
    num_workers = num_cores * num_subcores
    n_chunks = b_flat // (num_workers * CHUNK)

    mesh = plsc.VectorSubcoreMesh(core_axis_name="c", subcore_axis_name="s")

    @functools.partial(
        pl.kernel,
        mesh=mesh,
        out_type=jax.ShapeDtypeStruct((b_flat, EMBED_DIM), jnp.float32),
        scratch_types=[
            pltpu.VMEM((n_chunks * CHUNK,), jnp.int32),
            pltpu.VMEM((CHUNK, EMBED_DIM), jnp.float32),
            pltpu.SemaphoreType.DMA,
        ],
        compiler_params=pltpu.CompilerParams(use_tc_tiling_on_sc=False),
    )
    def emb_kernel(idx_hbm, table_hbm, out_hbm, idx_v, rows_v, gsem):
        wid = lax.axis_index("s") * num_cores + lax.axis_index("c")
        base = wid * n_chunks * CHUNK
        pltpu.sync_copy(idx_hbm.at[pl.ds(base, n_chunks * CHUNK)], idx_v)

        def body(j, carry):
            pltpu.async_copy(
                table_hbm.at[idx_v.at[pl.ds(j * CHUNK, CHUNK)]], rows_v, gsem
            ).wait()
            pltpu.sync_copy(
                rows_v, out_hbm.at[pl.ds(base + j * CHUNK, CHUNK)]
            )
            return carry

        lax.fori_loop(0, n_chunks, body, 0)

    return emb_kernel


def kernel(x, table):
    batch, hist = x.shape
    b_flat = batch * hist
    idx = x.reshape(b_flat)
    out = _make_kernel(b_flat)(idx, table)
    return out.reshape(batch, hist, EMBED_DIM)


# speedup vs baseline: 4.6561x; 1.1392x over previous
import functools

import jax
import jax.numpy as jnp
from jax import lax
from jax.experimental import pallas as pl
from jax.experimental.pallas import tpu as pltpu
from jax.experimental.pallas import tpu_sc as plsc

EMBED_DIM = 64
CHUNK = 128
NBUF = 5


@functools.cache
def _make_kernel(b_flat):
    info = plsc.get_sparse_core_info()
    num_cores, num_subcores = info.num_cores, info.num_subcores
    num_workers = num_cores * num_subcores
    n_chunks = b_flat // (num_workers * CHUNK)
    assert n_chunks % NBUF == 0
    n_groups = n_chunks // NBUF

    mesh = plsc.VectorSubcoreMesh(core_axis_name="c", subcore_axis_name="s")

    @functools.partial(
        pl.kernel,
        mesh=mesh,
        out_type=jax.ShapeDtypeStruct((b_flat, EMBED_DIM), jnp.float32),
        scratch_types=[
            pltpu.VMEM((n_chunks * CHUNK,), jnp.int32),
            pltpu.VMEM((NBUF, CHUNK, EMBED_DIM), jnp.float32),
        ]
        + [pltpu.SemaphoreType.DMA] * (2 * NBUF),
        compiler_params=pltpu.CompilerParams(use_tc_tiling_on_sc=False),
    )
    def emb_kernel(idx_hbm, table_hbm, out_hbm, idx_v, rows_v, *sems):
        gsem, ssem = sems[:NBUF], sems[NBUF:]
        wid = lax.axis_index("s") * num_cores + lax.axis_index("c")
        base = wid * n_chunks * CHUNK
        pltpu.sync_copy(idx_hbm.at[pl.ds(base, n_chunks * CHUNK)], idx_v)

        def gather(j, b):
            return pltpu.make_async_copy(
                table_hbm.at[idx_v.at[pl.ds(j * CHUNK, CHUNK)]],
                rows_v.at[b],
                gsem[b],
            )

        def scatter(j, b):
            return pltpu.make_async_copy(
                rows_v.at[b], out_hbm.at[pl.ds(base + j * CHUNK, CHUNK)], ssem[b]
            )

        for b in range(NBUF):
            gather(b, b).start()

        def body(g, carry):
            j0 = g * NBUF
            for b in range(NBUF):
                gather(j0 + b, b).wait()
                scatter(j0 + b, b).start()
            for b in range(NBUF):
                scatter(j0 + b, b).wait()
                gather(j0 + NBUF + b, b).start()
            return carry

        lax.fori_loop(0, n_groups - 1, body, 0)

        j0 = (n_groups - 1) * NBUF
        for b in range(NBUF):
            gather(j0 + b, b).wait()
            scatter(j0 + b, b).start()
        for b in range(NBUF):
            scatter(j0 + b, b).wait()

    return emb_kernel


def kernel(x, table):
    batch, hist = x.shape
    b_flat = batch * hist
    idx = x.reshape(b_flat)
    out = _make_kernel(b_flat)(idx, table)
    return out.reshape(batch, hist, EMBED_DIM)
